# Initial kernel scaffold; baseline (speedup 1.0000x reference)
import jax
import jax.numpy as jnp
from jax.experimental import pallas as pl


def kernel(pos, normal, params, batch):
    raise NotImplementedError("write your pallas kernel here")


# speedup vs baseline: 1.3564x; 1.3564x over previous
import functools

import jax
import jax.numpy as jnp
from jax.experimental import pallas as pl
from jax.experimental.pallas import tpu as pltpu

B, P, K = 8, 2048, 30
K32 = 32
RB = 256
EB = 128
EPS = 1e-5


def _topk_body(xr_ref, xa_ref, idx_ref):
    xr = xr_ref[0]
    xa = xa_ref[0]
    d2r = jnp.sum(xr * xr, axis=1)
    d2a = jnp.sum(xa * xa, axis=1)
    cross = jax.lax.dot_general(xr, xa, (((1,), (1,)), ((), ())),
                                preferred_element_type=jnp.float32)
    D = d2r[:, None] + d2a[None, :] - 2.0 * cross
    lanes = jax.lax.broadcasted_iota(jnp.int32, D.shape, 1)
    kcol = jax.lax.broadcasted_iota(jnp.int32, (RB, K32), 1)

    def body(k, carry):
        D, idxm = carry
        m = jnp.min(D, axis=1)
        cand = jnp.where(D == m[:, None], lanes, P)
        am = jnp.min(cand, axis=1)
        idxm = jnp.where(kcol == k, am[:, None], idxm)
        D = jnp.where(lanes == am[:, None], jnp.float32(jnp.inf), D)
        return D, idxm

    idx0 = jnp.zeros((RB, K32), jnp.int32)
    _, idxm = jax.lax.fori_loop(0, K, body, (D, idx0))
    idx_ref[0] = idxm


def _topk(x):
    d = x.shape[-1]
    return pl.pallas_call(
        _topk_body,
        grid=(B, P // RB),
        in_specs=[
            pl.BlockSpec((1, RB, d), lambda b, r: (b, r, 0)),
            pl.BlockSpec((1, P, d), lambda b, r: (b, 0, 0)),
        ],
        out_specs=pl.BlockSpec((1, RB, K32), lambda b, r: (b, r, 0)),
        out_shape=jax.ShapeDtypeStruct((B, P, K32), jnp.int32),
    )(x, x)


def _edge_h1(x, xj, w1, b1):
    d = x.shape[-1]
    xi = jnp.broadcast_to(x[:, None, :], (EB, K32, d)).reshape(EB * K32, d)
    e = jnp.concatenate([xi, xj - xi], axis=1)
    z = jax.lax.dot_general(e, w1, (((1,), (0,)), ((), ())),
                            preferred_element_type=jnp.float32) + b1
    return jnp.maximum(z, 0.0)


def _kmask():
    return (jax.lax.broadcasted_iota(jnp.int32, (EB, K32, 1), 1)
            < K).reshape(EB * K32, 1)


def _p1_body(x_ref, xj_ref, w1_ref, b1_ref, s_ref, q_ref):
    h = _edge_h1(x_ref[0], xj_ref[0], w1_ref[...], b1_ref[...])
    hm = jnp.where(_kmask(), h, 0.0)
    s_ref[0, 0, 0] = jnp.sum(hm, axis=0)
    q_ref[0, 0, 0] = jnp.sum(hm * hm, axis=0)


def _pass1(x, xjf, W1p, b1):
    nb = P // EB
    d = x.shape[-1]
    return pl.pallas_call(
        _p1_body,
        grid=(B, nb),
        in_specs=[
            pl.BlockSpec((1, EB, d), lambda b, r: (b, r, 0)),
            pl.BlockSpec((1, EB * K32, d), lambda b, r: (b, r, 0)),
            pl.BlockSpec((2 * d, 64), lambda b, r: (0, 0)),
            pl.BlockSpec((1, 64), lambda b, r: (0, 0)),
        ],
        out_specs=[
            pl.BlockSpec((1, 1, 1, 64), lambda b, r: (b, r, 0, 0)),
            pl.BlockSpec((1, 1, 1, 64), lambda b, r: (b, r, 0, 0)),
        ],
        out_shape=[
            jax.ShapeDtypeStruct((B, nb, 1, 64), jnp.float32),
            jax.ShapeDtypeStruct((B, nb, 1, 64), jnp.float32),
        ],
    )(x, xjf, W1p, b1[None, :])


def _p2_body(x_ref, xj_ref, w1_ref, b1_ref, mu_ref, sv_ref, g_ref, be_ref,
             w2_ref, b2_ref, mx_ref, mn_ref, s_ref, q_ref):
    h = _edge_h1(x_ref[0], xj_ref[0], w1_ref[...], b1_ref[...])
    hn = (h - mu_ref[...]) / sv_ref[...] * g_ref[...] + be_ref[...]
    z = jax.lax.dot_general(hn, w2_ref[...], (((1,), (0,)), ((), ())),
                            preferred_element_type=jnp.float32) + b2_ref[...]
    h2 = jnp.maximum(z, 0.0)
    km = _kmask()
    h2m = jnp.where(km, h2, 0.0)
    s_ref[0, 0, 0] = jnp.sum(h2m, axis=0)
    q_ref[0, 0, 0] = jnp.sum(h2m * h2m, axis=0)
    h2r = h2.reshape(EB, K32, 64)
    km3 = jax.lax.broadcasted_iota(jnp.int32, (EB, K32, 1), 1) < K
    mx_ref[0] = jnp.max(jnp.where(km3, h2r, -jnp.inf), axis=1)
    mn_ref[0] = jnp.min(jnp.where(km3, h2r, jnp.inf), axis=1)


def _pass2(x, xjf, W1p, b1, mu1, sv1, g1, be1, W2, b2):
    nb = P // EB
    d = x.shape[-1]
    row = lambda v: v[None, :]
    return pl.pallas_call(
        _p2_body,
        grid=(B, nb),
        in_specs=[
            pl.BlockSpec((1, EB, d), lambda b, r: (b, r, 0)),
            pl.BlockSpec((1, EB * K32, d), lambda b, r: (b, r, 0)),
            pl.BlockSpec((2 * d, 64), lambda b, r: (0, 0)),
            pl.BlockSpec((1, 64), lambda b, r: (0, 0)),
            pl.BlockSpec((1, 64), lambda b, r: (0, 0)),
            pl.BlockSpec((1, 64), lambda b, r: (0, 0)),
            pl.BlockSpec((1, 64), lambda b, r: (0, 0)),
            pl.BlockSpec((1, 64), lambda b, r: (0, 0)),
            pl.BlockSpec((64, 64), lambda b, r: (0, 0)),
            pl.BlockSpec((1, 64), lambda b, r: (0, 0)),
        ],
        out_specs=[
            pl.BlockSpec((1, EB, 64), lambda b, r: (b, r, 0)),
            pl.BlockSpec((1, EB, 64), lambda b, r: (b, r, 0)),
            pl.BlockSpec((1, 1, 1, 64), lambda b, r: (b, r, 0, 0)),
            pl.BlockSpec((1, 1, 1, 64), lambda b, r: (b, r, 0, 0)),
        ],
        out_shape=[
            jax.ShapeDtypeStruct((B, P, 64), jnp.float32),
            jax.ShapeDtypeStruct((B, P, 64), jnp.float32),
            jax.ShapeDtypeStruct((B, nb, 1, 64), jnp.float32),
            jax.ShapeDtypeStruct((B, nb, 1, 64), jnp.float32),
        ],
    )(x, xjf, W1p, row(b1), row(mu1), row(sv1), row(g1), row(be1), W2, row(b2))


def _fin_body(x1_ref, x2_ref, x3_ref, w_ref, b_ref, h_ref, s_ref, q_ref):
    cat = jnp.concatenate([x1_ref[...], x2_ref[...], x3_ref[...]], axis=1)
    z = jax.lax.dot_general(cat, w_ref[...], (((1,), (0,)), ((), ())),
                            preferred_element_type=jnp.float32) + b_ref[...]
    h = jnp.maximum(z, 0.0)
    h_ref[...] = h
    s_ref[0] = jnp.sum(h, axis=0, keepdims=True)
    q_ref[0] = jnp.sum(h * h, axis=0, keepdims=True)


def _final(x1, x2, x3, W, b):
    n = x1.shape[0]
    blk = 512
    nb = n // blk
    return pl.pallas_call(
        _fin_body,
        grid=(nb,),
        in_specs=[
            pl.BlockSpec((blk, 64), lambda i: (i, 0)),
            pl.BlockSpec((blk, 64), lambda i: (i, 0)),
            pl.BlockSpec((blk, 64), lambda i: (i, 0)),
            pl.BlockSpec((192, 1024), lambda i: (0, 0)),
            pl.BlockSpec((1, 1024), lambda i: (0, 0)),
        ],
        out_specs=[
            pl.BlockSpec((blk, 1024), lambda i: (i, 0)),
            pl.BlockSpec((1, 1, 1024), lambda i: (i, 0, 0)),
            pl.BlockSpec((1, 1, 1024), lambda i: (i, 0, 0)),
        ],
        out_shape=[
            jax.ShapeDtypeStruct((n, 1024), jnp.float32),
            jax.ShapeDtypeStruct((nb, 1, 1024), jnp.float32),
            jax.ShapeDtypeStruct((nb, 1, 1024), jnp.float32),
        ],
    )(x1, x2, x3, W, b[None, :])


def _pad_w1(W1, d, dp):
    if d == dp:
        return W1
    z = jnp.zeros((dp - d, W1.shape[1]), W1.dtype)
    return jnp.concatenate([W1[:d], z, W1[d:], z], axis=0)


def _edge_conv(x, p, d):
    dp = x.shape[-1]
    idx = _topk(x)
    xj = jax.vmap(lambda t, i: t[i])(x, idx)
    xjf = xj.reshape(B, P * K32, dp)
    W1p = _pad_w1(p['W1'], d, dp)
    s1p, q1p = _pass1(x, xjf, W1p, p['b1'])
    nedge = B * P * K
    mu1 = jnp.sum(s1p, axis=(0, 1, 2)) / nedge
    var1 = jnp.sum(q1p, axis=(0, 1, 2)) / nedge - mu1 * mu1
    sv1 = jnp.sqrt(var1 + EPS)
    Mx, Mn, s2p, q2p = _pass2(x, xjf, W1p, p['b1'], mu1, sv1,
                              p['g1'], p['be1'], p['W2'], p['b2'])
    mu2 = jnp.sum(s2p, axis=(0, 1, 2)) / nedge
    var2 = jnp.sum(q2p, axis=(0, 1, 2)) / nedge - mu2 * mu2
    sv2 = jnp.sqrt(var2 + EPS)
    g2, be2 = p['g2'], p['be2']
    out = jnp.where(g2 >= 0.0,
                    (Mx - mu2) / sv2 * g2 + be2,
                    (Mn - mu2) / sv2 * g2 + be2)
    return out


def _xla_edge_mlp(x, idx, p):
    xj = jax.vmap(lambda xb, ib: xb[ib])(x, idx)
    xi = jnp.broadcast_to(x[:, :, None, :], xj.shape)
    e = jnp.concatenate([xi, xj - xi], axis=-1).reshape(-1, 2 * x.shape[-1])
    h = jax.nn.relu(e @ p['W1'] + p['b1'])
    mu = jnp.mean(h, axis=0)
    var = jnp.var(h, axis=0)
    hn = (h - mu) / jnp.sqrt(var + EPS) * p['g1'] + p['be1']
    h2 = jax.nn.relu(hn @ p['W2'] + p['b2'])
    mu2 = jnp.mean(h2, axis=0)
    var2 = jnp.var(h2, axis=0)
    h2n = (h2 - mu2) / jnp.sqrt(var2 + EPS) * p['g2'] + p['be2']
    return jnp.max(h2n.reshape(B, P, K, 64), axis=2)


def kernel(pos, normal, params, batch):
    x0 = jnp.concatenate([pos, normal], axis=-1).reshape(B, P, 6)
    x0p = jnp.concatenate([x0, jnp.zeros((B, P, 2), jnp.float32)], axis=-1)
    idx1 = _topk(x0p)[..., :K]
    x1 = _xla_edge_mlp(x0, idx1, params['c1'])
    idx2 = _topk(x1)[..., :K]
    x2 = _xla_edge_mlp(x1, idx2, params['c2'])
    x3 = _edge_conv(x2, params['c3'], 64)
    lp = params['lin']
    h, sp, qp = _final(x1.reshape(B * P, 64), x2.reshape(B * P, 64),
                       x3.reshape(B * P, 64), lp['W'], lp['b'])
    n = B * P
    mu = jnp.sum(sp, axis=(0, 1)) / n
    var = jnp.sum(qp, axis=(0, 1)) / n - mu * mu
    out = (h - mu) / jnp.sqrt(var + EPS) * lp['g'] + lp['be']
    return out
